# baseline (speedup 1.0000x reference)
import jax
import jax.numpy as jnp
from jax.experimental import pallas as pl
from jax.experimental.pallas import tpu as pltpu

N = 256
EMB = 64
RBLK = 256


def _fused_kernel(skill_ref, adj_ref, emb_ref, W_ref, Wh_ref, Wx_ref,
                  bias_ref, bx_ref, out_ref,
                  hidden_ref, A_ref, B_ref, agg_ref):
    s = pl.program_id(0)
    b = pl.program_id(1)
    rb = pl.program_id(2)
    ns = pl.num_programs(0)
    bs = pl.num_programs(1)

    @pl.when(jnp.logical_and(s == 0, jnp.logical_and(b == 0, rb == 0)))
    def _init():
        hidden_ref[...] = jnp.zeros_like(hidden_ref)

    @pl.when(jnp.logical_and(s > 0, jnp.logical_and(b == 0, rb == 0)))
    def _recur():
        sm1 = s - 1
        agg2 = agg_ref[...].reshape(bs * N, EMB)
        hg = jnp.dot(agg2, Wh_ref[...],
                     preferred_element_type=jnp.float32)
        inp = jnp.concatenate(
            [emb_ref[0, skill_ref[bb, sm1], :].reshape(1, EMB)
             for bb in range(8)], axis=0)
        xg = (jnp.dot(inp, Wx_ref[...],
                      preferred_element_type=jnp.float32)
              + bx_ref[0:1, :])
        hg3 = hg.reshape(bs, N, 3 * EMB)
        g = hg3 + xg[:, None, :]
        r = jax.nn.sigmoid(g[..., :EMB])
        i = jax.nn.sigmoid(g[..., EMB:2 * EMB])
        n = jnp.tanh(xg[:, None, 2 * EMB:] + r * hg3[..., 2 * EMB:])
        h_new = (1.0 - i) * n + i * hidden_ref[...]
        hidden_ref[...] = h_new
        h2 = h_new.reshape(bs * N, EMB)
        p = jax.nn.relu(
            jnp.dot(h2, W_ref[3], preferred_element_type=jnp.float32)
            + bias_ref[2:3, :])
        p = jax.nn.relu(
            jnp.dot(p, W_ref[4], preferred_element_type=jnp.float32)
            + bias_ref[3:4, :])
        p = jnp.dot(p, W_ref[5], preferred_element_type=jnp.float32) \
            + bias_ref[4:5, :]
        out_ref[sm1] = p.reshape(bs, N, EMB)

    @pl.when(jnp.logical_and(s < ns - 1, jnp.logical_and(b == 0, rb == 0)))
    def _project():
        h2 = hidden_ref[...].reshape(bs * N, EMB)
        A_ref[...] = jnp.dot(h2, W_ref[0],
                             preferred_element_type=jnp.float32).reshape(
                                 bs, N, EMB)
        B_ref[...] = (jnp.dot(h2, W_ref[1],
                              preferred_element_type=jnp.float32)
                      + bias_ref[0:1, :]).reshape(bs, N, EMB)

    @pl.when(s < ns - 1)
    def _edges():
        b2 = bias_ref[1:2, :]
        W2 = W_ref[2].astype(jnp.bfloat16)
        A_b = A_ref[b]
        CH = 16
        for c in range(RBLK // CH):
            r0 = rb * RBLK + c * CH
            B_c = B_ref[b, pl.ds(r0, CH), :]
            u = jnp.tanh(A_b[None, :, :] + B_c[:, None, :]).astype(
                jnp.bfloat16)
            m = jnp.tanh(
                jnp.dot(u.reshape(CH * N, EMB), W2,
                        preferred_element_type=jnp.float32) + b2)
            adj_c = adj_ref[0, 0, pl.ds(c * CH, CH), :]
            agg_ref[b, pl.ds(r0, CH), :] = jax.lax.dot_general(
                adj_c, m.reshape(CH, N, EMB),
                dimension_numbers=(((1,), (1,)), ((0,), (0,))),
                preferred_element_type=jnp.float32)


@jax.jit
def _run(adj, node_embeddings, skill_seq, W_stack, Wh_cat, Wx_cat,
         bias_stack, bx_cat):
    bs = skill_seq.shape[0]
    tm1 = skill_seq.shape[1] - 1
    nb = N // RBLK
    out = pl.pallas_call(
        _fused_kernel,
        grid=(tm1 + 1, bs, nb),
        in_specs=[
            pl.BlockSpec(memory_space=pltpu.SMEM),
            pl.BlockSpec((1, 1, RBLK, N),
                         lambda s, b, rb: (0, b, rb, 0)),
            pl.BlockSpec((1, N, EMB), lambda s, b, rb: (0, 0, 0)),
            pl.BlockSpec((6, EMB, EMB), lambda s, b, rb: (0, 0, 0)),
            pl.BlockSpec((EMB, 3 * EMB), lambda s, b, rb: (0, 0)),
            pl.BlockSpec((EMB, 3 * EMB), lambda s, b, rb: (0, 0)),
            pl.BlockSpec((5, EMB), lambda s, b, rb: (0, 0)),
            pl.BlockSpec((1, 3 * EMB), lambda s, b, rb: (0, 0)),
        ],
        out_specs=pl.BlockSpec((tm1, bs, N, EMB),
                               lambda s, b, rb: (0, 0, 0, 0)),
        out_shape=jax.ShapeDtypeStruct((tm1, bs, N, EMB), jnp.float32),
        scratch_shapes=[
            pltpu.VMEM((bs, N, EMB), jnp.float32),
            pltpu.VMEM((bs, N, EMB), jnp.float32),
            pltpu.VMEM((bs, N, EMB), jnp.float32),
            pltpu.VMEM((bs, N, EMB), jnp.float32),
        ],
        compiler_params=pltpu.CompilerParams(
            dimension_semantics=("arbitrary", "arbitrary", "arbitrary")),
    )(skill_seq, adj, node_embeddings, W_stack, Wh_cat, Wx_cat,
      bias_stack, bx_cat)
    return jnp.transpose(out, (1, 2, 0, 3))


def kernel(adj, node_embeddings, W_msg1, b_msg1, W_msg2, b_msg2, W_hr, W_hi,
           W_hh, W_ir, b_ir, W_ii, b_ii, W_in, b_in, W_o1, b_o1, W_o2, b_o2,
           W_o3, b_o3, skill_seq, problem_seq, time_seq, label_seq):
    W_stack = jnp.stack([
        W_msg1[:, :EMB].T,
        W_msg1[:, EMB:].T,
        W_msg2.T,
        W_o1.T,
        W_o2.T,
        W_o3.T,
    ])
    Wh_cat = jnp.concatenate(
        [W_hr.T, W_hi.T, W_hh.T], axis=1) * (1.0 / N)
    Wx_cat = jnp.concatenate([W_ir.T, W_ii.T, W_in.T], axis=1)
    bias_stack = jnp.stack([b_msg1, b_msg2, b_o1, b_o2, b_o3])
    bx_cat = jnp.concatenate([b_ir, b_ii, b_in]).reshape(1, 3 * EMB)
    return _run(adj, node_embeddings, skill_seq, W_stack, Wh_cat, Wx_cat,
                bias_stack, bx_cat)


# speedup vs baseline: 1.1440x; 1.1440x over previous
import jax
import jax.numpy as jnp
from jax.experimental import pallas as pl
from jax.experimental.pallas import tpu as pltpu

N = 256
EMB = 64
RBLK = 256


def _fused_kernel(skill_ref, adj_ref, emb_ref, W_ref, Wh_ref, Wx_ref,
                  bias_ref, bx_ref, out_ref,
                  hidden_ref, A_ref, B_ref, agg_ref):
    s = pl.program_id(0)
    b = pl.program_id(1)
    rb = pl.program_id(2)
    ns = pl.num_programs(0)
    bs = pl.num_programs(1)

    @pl.when(jnp.logical_and(s == 0, jnp.logical_and(b == 0, rb == 0)))
    def _init():
        hidden_ref[...] = jnp.zeros_like(hidden_ref)

    @pl.when(jnp.logical_and(s > 0, jnp.logical_and(b == 0, rb == 0)))
    def _recur():
        sm1 = s - 1
        agg2 = agg_ref[...].reshape(bs * N, EMB)
        hg = jnp.dot(agg2, Wh_ref[...],
                     preferred_element_type=jnp.float32)
        inp = jnp.concatenate(
            [emb_ref[0, skill_ref[bb, sm1], :].reshape(1, EMB)
             for bb in range(8)], axis=0)
        xg = (jnp.dot(inp, Wx_ref[...],
                      preferred_element_type=jnp.float32)
              + bx_ref[0:1, :])
        hg3 = hg.reshape(bs, N, 3 * EMB)
        g = hg3 + xg[:, None, :]
        r = jax.nn.sigmoid(g[..., :EMB])
        i = jax.nn.sigmoid(g[..., EMB:2 * EMB])
        n = jnp.tanh(xg[:, None, 2 * EMB:] + r * hg3[..., 2 * EMB:])
        h_new = (1.0 - i) * n + i * hidden_ref[...]
        hidden_ref[...] = h_new
        h2 = h_new.reshape(bs * N, EMB)
        p = jax.nn.relu(
            jnp.dot(h2, W_ref[3], preferred_element_type=jnp.float32)
            + bias_ref[2:3, :])
        p = jax.nn.relu(
            jnp.dot(p, W_ref[4], preferred_element_type=jnp.float32)
            + bias_ref[3:4, :])
        p = jnp.dot(p, W_ref[5], preferred_element_type=jnp.float32) \
            + bias_ref[4:5, :]
        out_ref[sm1] = p.reshape(bs, N, EMB)

    @pl.when(jnp.logical_and(s < ns - 1, jnp.logical_and(b == 0, rb == 0)))
    def _project():
        h2 = hidden_ref[...].reshape(bs * N, EMB)
        A_ref[...] = jnp.dot(h2, W_ref[0],
                             preferred_element_type=jnp.float32).reshape(
                                 bs, N, EMB)
        B_ref[...] = (jnp.dot(h2, W_ref[1],
                              preferred_element_type=jnp.float32)
                      + bias_ref[0:1, :]).reshape(bs, N, EMB)

    @pl.when(s < ns - 1)
    def _edges():
        b2 = bias_ref[1:2, :]
        W2 = W_ref[2].astype(jnp.bfloat16)
        A_b = A_ref[b]
        CH = 64
        for c in range(RBLK // CH):
            r0 = rb * RBLK + c * CH
            B_c = B_ref[b, pl.ds(r0, CH), :]
            u = jnp.tanh(A_b[None, :, :] + B_c[:, None, :]).astype(
                jnp.bfloat16)
            m = jnp.tanh(
                jnp.dot(u.reshape(CH * N, EMB), W2,
                        preferred_element_type=jnp.float32) + b2)
            adj_c = adj_ref[0, 0, pl.ds(c * CH, CH), :]
            agg_ref[b, pl.ds(r0, CH), :] = jax.lax.dot_general(
                adj_c, m.reshape(CH, N, EMB),
                dimension_numbers=(((1,), (1,)), ((0,), (0,))),
                preferred_element_type=jnp.float32)


@jax.jit
def _run(adj, node_embeddings, skill_seq, W_stack, Wh_cat, Wx_cat,
         bias_stack, bx_cat):
    bs = skill_seq.shape[0]
    tm1 = skill_seq.shape[1] - 1
    nb = N // RBLK
    out = pl.pallas_call(
        _fused_kernel,
        grid=(tm1 + 1, bs, nb),
        in_specs=[
            pl.BlockSpec(memory_space=pltpu.SMEM),
            pl.BlockSpec((1, 1, RBLK, N),
                         lambda s, b, rb: (0, b, rb, 0)),
            pl.BlockSpec((1, N, EMB), lambda s, b, rb: (0, 0, 0)),
            pl.BlockSpec((6, EMB, EMB), lambda s, b, rb: (0, 0, 0)),
            pl.BlockSpec((EMB, 3 * EMB), lambda s, b, rb: (0, 0)),
            pl.BlockSpec((EMB, 3 * EMB), lambda s, b, rb: (0, 0)),
            pl.BlockSpec((5, EMB), lambda s, b, rb: (0, 0)),
            pl.BlockSpec((1, 3 * EMB), lambda s, b, rb: (0, 0)),
        ],
        out_specs=pl.BlockSpec((tm1, bs, N, EMB),
                               lambda s, b, rb: (0, 0, 0, 0)),
        out_shape=jax.ShapeDtypeStruct((tm1, bs, N, EMB), jnp.float32),
        scratch_shapes=[
            pltpu.VMEM((bs, N, EMB), jnp.float32),
            pltpu.VMEM((bs, N, EMB), jnp.float32),
            pltpu.VMEM((bs, N, EMB), jnp.float32),
            pltpu.VMEM((bs, N, EMB), jnp.float32),
        ],
        compiler_params=pltpu.CompilerParams(
            dimension_semantics=("arbitrary", "arbitrary", "arbitrary")),
    )(skill_seq, adj, node_embeddings, W_stack, Wh_cat, Wx_cat,
      bias_stack, bx_cat)
    return jnp.transpose(out, (1, 2, 0, 3))


def kernel(adj, node_embeddings, W_msg1, b_msg1, W_msg2, b_msg2, W_hr, W_hi,
           W_hh, W_ir, b_ir, W_ii, b_ii, W_in, b_in, W_o1, b_o1, W_o2, b_o2,
           W_o3, b_o3, skill_seq, problem_seq, time_seq, label_seq):
    W_stack = jnp.stack([
        W_msg1[:, :EMB].T,
        W_msg1[:, EMB:].T,
        W_msg2.T,
        W_o1.T,
        W_o2.T,
        W_o3.T,
    ])
    Wh_cat = jnp.concatenate(
        [W_hr.T, W_hi.T, W_hh.T], axis=1) * (1.0 / N)
    Wx_cat = jnp.concatenate([W_ir.T, W_ii.T, W_in.T], axis=1)
    bias_stack = jnp.stack([b_msg1, b_msg2, b_o1, b_o2, b_o3])
    bx_cat = jnp.concatenate([b_ir, b_ii, b_in]).reshape(1, 3 * EMB)
    return _run(adj, node_embeddings, skill_seq, W_stack, Wh_cat, Wx_cat,
                bias_stack, bx_cat)


# speedup vs baseline: 1.1712x; 1.0238x over previous
import jax
import jax.numpy as jnp
from jax.experimental import pallas as pl
from jax.experimental.pallas import tpu as pltpu

N = 256
EMB = 64
RBLK = 256


def _fused_kernel(skill_ref, adj_ref, emb_ref, W_ref, Wh_ref, Wx_ref,
                  bias_ref, bx_ref, out_ref,
                  hidden_ref, A_ref, B_ref, agg_ref):
    s = pl.program_id(0)
    b = pl.program_id(1)
    rb = pl.program_id(2)
    ns = pl.num_programs(0)
    bs = pl.num_programs(1)

    @pl.when(jnp.logical_and(s == 0, jnp.logical_and(b == 0, rb == 0)))
    def _init():
        hidden_ref[...] = jnp.zeros_like(hidden_ref)

    @pl.when(jnp.logical_and(s > 0, jnp.logical_and(b == 0, rb == 0)))
    def _recur():
        sm1 = s - 1
        agg2 = agg_ref[...].reshape(bs * N, EMB)
        hg = jnp.dot(agg2, Wh_ref[...],
                     preferred_element_type=jnp.float32)
        inp = jnp.concatenate(
            [emb_ref[0, skill_ref[bb, sm1], :].reshape(1, EMB)
             for bb in range(8)], axis=0)
        xg = (jnp.dot(inp, Wx_ref[...],
                      preferred_element_type=jnp.float32)
              + bx_ref[0:1, :])
        hg3 = hg.reshape(bs, N, 3 * EMB)
        g = hg3 + xg[:, None, :]
        r = jax.nn.sigmoid(g[..., :EMB])
        i = jax.nn.sigmoid(g[..., EMB:2 * EMB])
        n = jnp.tanh(xg[:, None, 2 * EMB:] + r * hg3[..., 2 * EMB:])
        h_new = (1.0 - i) * n + i * hidden_ref[...]
        hidden_ref[...] = h_new
        h2 = h_new.reshape(bs * N, EMB)
        p = jax.nn.relu(
            jnp.dot(h2, W_ref[3], preferred_element_type=jnp.float32)
            + bias_ref[2:3, :])
        p = jax.nn.relu(
            jnp.dot(p, W_ref[4], preferred_element_type=jnp.float32)
            + bias_ref[3:4, :])
        p = jnp.dot(p, W_ref[5], preferred_element_type=jnp.float32) \
            + bias_ref[4:5, :]
        out_ref[sm1] = p.reshape(bs, N, EMB)

    @pl.when(jnp.logical_and(s < ns - 1, jnp.logical_and(b == 0, rb == 0)))
    def _project():
        h2 = hidden_ref[...].reshape(bs * N, EMB)
        A_ref[...] = jnp.dot(h2, W_ref[0],
                             preferred_element_type=jnp.float32).reshape(
                                 bs, N, EMB)
        B_ref[...] = (jnp.dot(h2, W_ref[1],
                              preferred_element_type=jnp.float32)
                      + bias_ref[0:1, :]).reshape(bs, N, EMB)

    @pl.when(s < ns - 1)
    def _edges():
        b2 = bias_ref[1:2, :]
        W2 = W_ref[2].astype(jnp.bfloat16)
        A_b = A_ref[b]
        CH = 128
        for c in range(RBLK // CH):
            r0 = rb * RBLK + c * CH
            B_c = B_ref[b, pl.ds(r0, CH), :]
            u = jnp.tanh(A_b[None, :, :] + B_c[:, None, :]).astype(
                jnp.bfloat16)
            m = jnp.tanh(
                jnp.dot(u.reshape(CH * N, EMB), W2,
                        preferred_element_type=jnp.float32) + b2)
            adj_c = adj_ref[0, 0, pl.ds(c * CH, CH), :]
            agg_ref[b, pl.ds(r0, CH), :] = jax.lax.dot_general(
                adj_c, m.reshape(CH, N, EMB),
                dimension_numbers=(((1,), (1,)), ((0,), (0,))),
                preferred_element_type=jnp.float32)


@jax.jit
def _run(adj, node_embeddings, skill_seq, W_stack, Wh_cat, Wx_cat,
         bias_stack, bx_cat):
    bs = skill_seq.shape[0]
    tm1 = skill_seq.shape[1] - 1
    nb = N // RBLK
    out = pl.pallas_call(
        _fused_kernel,
        grid=(tm1 + 1, bs, nb),
        in_specs=[
            pl.BlockSpec(memory_space=pltpu.SMEM),
            pl.BlockSpec((1, 1, RBLK, N),
                         lambda s, b, rb: (0, b, rb, 0)),
            pl.BlockSpec((1, N, EMB), lambda s, b, rb: (0, 0, 0)),
            pl.BlockSpec((6, EMB, EMB), lambda s, b, rb: (0, 0, 0)),
            pl.BlockSpec((EMB, 3 * EMB), lambda s, b, rb: (0, 0)),
            pl.BlockSpec((EMB, 3 * EMB), lambda s, b, rb: (0, 0)),
            pl.BlockSpec((5, EMB), lambda s, b, rb: (0, 0)),
            pl.BlockSpec((1, 3 * EMB), lambda s, b, rb: (0, 0)),
        ],
        out_specs=pl.BlockSpec((tm1, bs, N, EMB),
                               lambda s, b, rb: (0, 0, 0, 0)),
        out_shape=jax.ShapeDtypeStruct((tm1, bs, N, EMB), jnp.float32),
        scratch_shapes=[
            pltpu.VMEM((bs, N, EMB), jnp.float32),
            pltpu.VMEM((bs, N, EMB), jnp.float32),
            pltpu.VMEM((bs, N, EMB), jnp.float32),
            pltpu.VMEM((bs, N, EMB), jnp.float32),
        ],
        compiler_params=pltpu.CompilerParams(
            dimension_semantics=("arbitrary", "arbitrary", "arbitrary")),
    )(skill_seq, adj, node_embeddings, W_stack, Wh_cat, Wx_cat,
      bias_stack, bx_cat)
    return jnp.transpose(out, (1, 2, 0, 3))


def kernel(adj, node_embeddings, W_msg1, b_msg1, W_msg2, b_msg2, W_hr, W_hi,
           W_hh, W_ir, b_ir, W_ii, b_ii, W_in, b_in, W_o1, b_o1, W_o2, b_o2,
           W_o3, b_o3, skill_seq, problem_seq, time_seq, label_seq):
    W_stack = jnp.stack([
        W_msg1[:, :EMB].T,
        W_msg1[:, EMB:].T,
        W_msg2.T,
        W_o1.T,
        W_o2.T,
        W_o3.T,
    ])
    Wh_cat = jnp.concatenate(
        [W_hr.T, W_hi.T, W_hh.T], axis=1) * (1.0 / N)
    Wx_cat = jnp.concatenate([W_ir.T, W_ii.T, W_in.T], axis=1)
    bias_stack = jnp.stack([b_msg1, b_msg2, b_o1, b_o2, b_o3])
    bx_cat = jnp.concatenate([b_ir, b_ii, b_in]).reshape(1, 3 * EMB)
    return _run(adj, node_embeddings, skill_seq, W_stack, Wh_cat, Wx_cat,
                bias_stack, bx_cat)
